# baseline (speedup 1.0000x reference)
import functools

import jax
import jax.numpy as jnp
from jax import lax
from jax.experimental import pallas as pl
from jax.experimental.pallas import tpu as pltpu
from jax.experimental.pallas import tpu_sc as plsc

_NW = 32
_VOCAB = 1_000_000
_DIM = 64
_NBLK = _VOCAB // 128
_NBLK_ALL = _NBLK + 1
_TAIL_C = _NBLK
_NBUF_A = 6
_NBUF_B = 4


def _iota16():
    return lax.broadcasted_iota(jnp.int32, (16,), 0)


def _detrans_body(wt_hbm, tail_hbm, wrm_hbm, a_bufs, b_bufs, isems,
                  osems):
    n = _NBUF_A
    wid = lax.axis_index("s") * 2 + jnp.int32(lax.axis_index("c"))
    ngrp_total = -(-_NBLK_ALL // n)
    per = ngrp_total // _NW
    extra = ngrp_total - per * _NW
    ngrp = per + (wid < extra).astype(jnp.int32)
    start = n * (wid * per + jnp.minimum(wid, extra))
    cnt = n * ngrp
    iota = _iota16()

    def fire_in(c, par):
        @pl.when(c < _TAIL_C)
        def _():
            off = pl.multiple_of(c * 128, 128)
            for r in range(8):
                pltpu.async_copy(
                    wt_hbm.at[pl.ds(8 * r, 8), pl.ds(off, 128)],
                    a_bufs.at[par].at[pl.ds(8 * r, 8), :], isems[par])

    def wait_in(c, par):
        @pl.when(c < _TAIL_C)
        def _():
            off = pl.multiple_of(c * 128, 128)
            pltpu.make_async_copy(wt_hbm.at[:, pl.ds(off, 128)],
                                  a_bufs.at[par], isems[par]).wait()

    def fire_out(c, par):
        @pl.when(c < _TAIL_C)
        def _():
            off = pl.multiple_of(c * 64, 64)
            pltpu.async_copy(b_bufs.at[par], wrm_hbm.at[pl.ds(off, 64), :],
                             osems[par])

    def wait_out(c, par):
        @pl.when(c < _TAIL_C)
        def _():
            off = pl.multiple_of(c * 64, 64)
            pltpu.make_async_copy(b_bufs.at[par],
                                  wrm_hbm.at[pl.ds(off, 64), :],
                                  osems[par]).wait()

    def shuffle(c, par):
        @pl.when(c < _TAIL_C)
        def _():
            zero = iota * 0
            rows = [iota + 16 * kk for kk in range(4)]

            def pstep(p2, carry):
                ce, co = carry
                vecs = []
                for u in range(2):
                    cu_e = ce + 2 * u
                    cu_o = co + 2 * u
                    for k in range(8):
                        col = cu_e if k < 4 else cu_o
                        vecs.append(plsc.load_gather(a_bufs.at[par],
                                                     [rows[k % 4], col]))
                for u in range(2):
                    p = 2 * p2 + u
                    for k in range(8):
                        b_bufs[par, p, pl.ds(16 * k, 16)] = vecs[8 * u + k]
                return (ce + 4, co + 4)

            lax.fori_loop(0, 32, pstep, (zero, zero + 1))

    @pl.when(wid == 31)
    def _():
        stage = b_bufs.at[0].at[pl.ds(0, 32), :]
        pltpu.sync_copy(tail_hbm, stage)
        pltpu.sync_copy(stage, wrm_hbm.at[pl.ds(_TAIL_C * 64, 32), :])

    for par in range(n):
        fire_in(start + par, par)

    def group(g, carry):
        for par in range(n):
            t = n * g + par
            c = start + t
            wait_in(c, par)

            @pl.when(g >= 1)
            def _():
                wait_out(c - n, par)

            shuffle(c, par)
            fire_out(c, par)

            @pl.when(g < ngrp - 1)
            def _():
                fire_in(c + n, par)

        return carry

    lax.fori_loop(0, ngrp, group, 0)

    for par in range(n):
        wait_out(start + cnt - n + par, par)


def _lookup_body(idx_hbm, wrm_hbm, out_hbm, idx_v, pair_v, parm_v, g_bufs,
                 o_bufs, gsems, osems):
    wid = lax.axis_index("s") * 2 + lax.axis_index("c")
    iota = _iota16()
    iota50 = iota * 50

    pltpu.sync_copy(idx_hbm.at[pl.ds(wid * 25600, 25600)], idx_v)

    def extract_and_fire(t, par):
        jbp = t // 50
        h = t - jbp * 50
        pos = iota50 + (128 * jbp * 50 + h)
        vs = []
        for k in range(8):
            vs.append(plsc.load_gather(idx_v, [pos]))
            if k < 7:
                pos = pos + 800
        for k in range(8):
            pair_v[par, pl.ds(16 * k, 16)] = lax.shift_right_logical(vs[k], 1)
            parm_v[par, pl.ds(16 * k, 16)] = (vs[k] & 1) * 64
        pltpu.async_copy(wrm_hbm.at[pair_v.at[par]], g_bufs.at[par],
                         gsems[par])

    def wait_gather(par):
        pltpu.make_async_copy(wrm_hbm.at[pair_v.at[par]], g_bufs.at[par],
                              gsems[par]).wait()

    def out_desc(t, par):
        jbp = t // 50
        h = t - jbp * 50
        off = pl.multiple_of(wid * 512 + jbp * 128, 128)
        return pltpu.make_async_copy(o_bufs.at[par],
                                     out_hbm.at[h, :, pl.ds(off, 128)],
                                     osems[par])

    def fire_out(t, par):
        jbp = t // 50
        h = t - jbp * 50
        off = pl.multiple_of(wid * 512 + jbp * 128, 128)
        for r in range(8):
            pltpu.async_copy(o_bufs.at[par].at[pl.ds(8 * r, 8), :],
                             out_hbm.at[h, pl.ds(8 * r, 8), pl.ds(off, 128)],
                             osems[par])

    def shuffle(par):
        rows = [iota + 16 * kk for kk in range(8)]
        cols0 = tuple(parm_v[par, pl.ds(16 * kk, 16)] for kk in range(8))

        def dstep(d2, cols):
            vecs = []
            for u in range(2):
                for k in range(8):
                    vecs.append(plsc.load_gather(g_bufs.at[par],
                                                 [rows[k], cols[k] + u]))
            for u in range(2):
                d = 2 * d2 + u
                for k in range(8):
                    o_bufs[par, d, pl.ds(16 * k, 16)] = vecs[8 * u + k]
            return tuple(ck + 2 for ck in cols)

        lax.fori_loop(0, 32, dstep, cols0)

    nb = _NBUF_B
    ng = 200 // nb
    for par in range(nb):
        extract_and_fire(jnp.int32(par), par)

    def group(g, carry):
        for par in range(nb):
            t = nb * g + par
            wait_gather(par)

            @pl.when(g >= 1)
            def _():
                out_desc(t - nb, par).wait()

            shuffle(par)
            fire_out(t, par)

            @pl.when(g < ng - 1)
            def _():
                extract_and_fire(t + nb, par)

        return carry

    lax.fori_loop(0, ng, group, 0)

    for par in range(nb):
        out_desc(200 - nb + par, par).wait()


@jax.jit
def kernel(x, weight):
    batch, hist = x.shape
    vocab, dim = weight.shape
    idx = x.astype(jnp.int32).reshape(batch * hist)
    wt = weight.T

    mesh = plsc.VectorSubcoreMesh(core_axis_name="c", subcore_axis_name="s")
    cp = pltpu.CompilerParams(use_tc_tiling_on_sc=True,
                              needs_layout_passes=False)

    wrm = pl.kernel(
        _detrans_body,
        out_type=jax.ShapeDtypeStruct((vocab // 2, 2 * dim), jnp.float32),
        mesh=mesh,
        scratch_types=[
            pltpu.VMEM((_NBUF_A, dim, 128), jnp.float32),
            pltpu.VMEM((_NBUF_A, dim, 128), jnp.float32),
            [pltpu.SemaphoreType.DMA] * _NBUF_A,
            [pltpu.SemaphoreType.DMA] * _NBUF_A,
        ],
        compiler_params=cp,
    )(wt, weight[_TAIL_C * 128:].reshape(32, 128))

    out5 = pl.kernel(
        _lookup_body,
        out_type=jax.ShapeDtypeStruct((hist, dim, batch), jnp.float32),
        mesh=mesh,
        scratch_types=[
            pltpu.VMEM((25600,), jnp.int32),
            pltpu.VMEM((_NBUF_B, 128), jnp.int32),
            pltpu.VMEM((_NBUF_B, 128), jnp.int32),
            pltpu.VMEM((_NBUF_B, 128, 128), jnp.float32),
            pltpu.VMEM((_NBUF_B, dim, 128), jnp.float32),
            [pltpu.SemaphoreType.DMA] * _NBUF_B,
            [pltpu.SemaphoreType.DMA] * _NBUF_B,
        ],
        compiler_params=cp,
    )(idx, wrm)

    return out5.transpose(2, 0, 1)


# speedup vs baseline: 1.9049x; 1.9049x over previous
import functools

import jax
import jax.numpy as jnp
from jax import lax
from jax.experimental import pallas as pl
from jax.experimental.pallas import tpu as pltpu
from jax.experimental.pallas import tpu_sc as plsc

_NW = 32
_VOCAB = 1_000_000
_DIM = 64
_NBLK = _VOCAB // 128
_NBLK_ALL = _NBLK + 1
_TAIL_C = _NBLK
_NBUF_A = 6
_NBUF_B = 4


def _iota16():
    return lax.broadcasted_iota(jnp.int32, (16,), 0)


def _detrans_body(wt_hbm, tail_hbm, wrm_hbm, a_bufs, b_bufs, isems,
                  osems):
    n = _NBUF_A
    wid = lax.axis_index("s") * 2 + jnp.int32(lax.axis_index("c"))
    ngrp_total = -(-_NBLK_ALL // n)
    per = ngrp_total // _NW
    extra = ngrp_total - per * _NW
    ngrp = per + (wid < extra).astype(jnp.int32)
    start = n * (wid * per + jnp.minimum(wid, extra))
    cnt = n * ngrp
    iota = _iota16()

    def fire_in(c, par):
        @pl.when(c < _TAIL_C)
        def _():
            off = pl.multiple_of(c * 128, 128)
            for r in range(8):
                pltpu.async_copy(
                    wt_hbm.at[pl.ds(8 * r, 8), pl.ds(off, 128)],
                    a_bufs.at[par].at[pl.ds(8 * r, 8), :], isems[par])

    def wait_in(c, par):
        @pl.when(c < _TAIL_C)
        def _():
            off = pl.multiple_of(c * 128, 128)
            pltpu.make_async_copy(wt_hbm.at[:, pl.ds(off, 128)],
                                  a_bufs.at[par], isems[par]).wait()

    def fire_out(c, par):
        @pl.when(c < _TAIL_C)
        def _():
            off = pl.multiple_of(c * 64, 64)
            pltpu.async_copy(b_bufs.at[par], wrm_hbm.at[pl.ds(off, 64), :],
                             osems[par])

    def wait_out(c, par):
        @pl.when(c < _TAIL_C)
        def _():
            off = pl.multiple_of(c * 64, 64)
            pltpu.make_async_copy(b_bufs.at[par],
                                  wrm_hbm.at[pl.ds(off, 64), :],
                                  osems[par]).wait()

    def shuffle(c, par):
        @pl.when(c < _TAIL_C)
        def _():
            def sstep(sh, carry):
                t = (iota + sh) & 15
                for m0 in range(0, 128, 16):
                    mv = m0 + t
                    rows_b = lax.shift_right_logical(mv, 1)
                    colb = (mv & 1) * 64
                    for d0 in range(0, 64, 16):
                        dv = d0 + iota
                        vec = plsc.load_gather(a_bufs.at[par], [dv, mv])
                        plsc.store_scatter(b_bufs.at[par], [rows_b, colb + dv],
                                           vec)
                return carry

            lax.fori_loop(0, 16, sstep, 0)

    @pl.when(wid == 31)
    def _():
        stage = b_bufs.at[0].at[pl.ds(0, 32), :]
        pltpu.sync_copy(tail_hbm, stage)
        pltpu.sync_copy(stage, wrm_hbm.at[pl.ds(_TAIL_C * 64, 32), :])

    for par in range(n):
        fire_in(start + par, par)

    def group(g, carry):
        for par in range(n):
            t = n * g + par
            c = start + t
            wait_in(c, par)

            @pl.when(g >= 1)
            def _():
                wait_out(c - n, par)

            shuffle(c, par)
            fire_out(c, par)

            @pl.when(g < ngrp - 1)
            def _():
                fire_in(c + n, par)

        return carry

    lax.fori_loop(0, ngrp, group, 0)

    for par in range(n):
        wait_out(start + cnt - n + par, par)


def _lookup_body(idx_hbm, wrm_hbm, out_hbm, idx_v, pair_v, parm_v, g_bufs,
                 o_bufs, gsems, osems):
    wid = lax.axis_index("s") * 2 + lax.axis_index("c")
    iota = _iota16()
    iota50 = iota * 50

    pltpu.sync_copy(idx_hbm.at[pl.ds(wid * 25600, 25600)], idx_v)

    def extract_and_fire(t, par):
        jbp = t // 50
        h = t - jbp * 50
        pos = iota50 + (128 * jbp * 50 + h)
        vs = []
        for k in range(8):
            vs.append(plsc.load_gather(idx_v, [pos]))
            if k < 7:
                pos = pos + 800
        for k in range(8):
            pair_v[par, pl.ds(16 * k, 16)] = lax.shift_right_logical(vs[k], 1)
            parm_v[par, pl.ds(16 * k, 16)] = (vs[k] & 1) * 64
        pltpu.async_copy(wrm_hbm.at[pair_v.at[par]], g_bufs.at[par],
                         gsems[par])

    def wait_gather(par):
        pltpu.make_async_copy(wrm_hbm.at[pair_v.at[par]], g_bufs.at[par],
                              gsems[par]).wait()

    def out_desc(t, par):
        jbp = t // 50
        h = t - jbp * 50
        off = pl.multiple_of(wid * 512 + jbp * 128, 128)
        return pltpu.make_async_copy(o_bufs.at[par],
                                     out_hbm.at[h, :, pl.ds(off, 128)],
                                     osems[par])

    def fire_out(t, par):
        jbp = t // 50
        h = t - jbp * 50
        off = pl.multiple_of(wid * 512 + jbp * 128, 128)
        for r in range(8):
            pltpu.async_copy(o_bufs.at[par].at[pl.ds(8 * r, 8), :],
                             out_hbm.at[h, pl.ds(8 * r, 8), pl.ds(off, 128)],
                             osems[par])

    def shuffle(par):
        def sstep(sh, carry):
            t = (iota + sh) & 15
            for b0 in range(0, 128, 16):
                bbv = b0 + t
                pv = plsc.load_gather(parm_v.at[par], [bbv])
                for d0 in range(0, 64, 16):
                    dv = d0 + iota
                    vec = plsc.load_gather(g_bufs.at[par], [bbv, pv + dv])
                    plsc.store_scatter(o_bufs.at[par], [dv, bbv], vec)
            return carry

        lax.fori_loop(0, 16, sstep, 0)

    nb = _NBUF_B
    ng = 200 // nb
    for par in range(nb):
        extract_and_fire(jnp.int32(par), par)

    def group(g, carry):
        for par in range(nb):
            t = nb * g + par
            wait_gather(par)

            @pl.when(g >= 1)
            def _():
                out_desc(t - nb, par).wait()

            shuffle(par)
            fire_out(t, par)

            @pl.when(g < ng - 1)
            def _():
                extract_and_fire(t + nb, par)

        return carry

    lax.fori_loop(0, ng, group, 0)

    for par in range(nb):
        out_desc(200 - nb + par, par).wait()


@jax.jit
def kernel(x, weight):
    batch, hist = x.shape
    vocab, dim = weight.shape
    idx = x.astype(jnp.int32).reshape(batch * hist)
    wt = weight.T

    mesh = plsc.VectorSubcoreMesh(core_axis_name="c", subcore_axis_name="s")
    cp = pltpu.CompilerParams(use_tc_tiling_on_sc=True,
                              needs_layout_passes=False)

    wrm = pl.kernel(
        _detrans_body,
        out_type=jax.ShapeDtypeStruct((vocab // 2, 2 * dim), jnp.float32),
        mesh=mesh,
        scratch_types=[
            pltpu.VMEM((_NBUF_A, dim, 128), jnp.float32),
            pltpu.VMEM((_NBUF_A, dim, 128), jnp.float32),
            [pltpu.SemaphoreType.DMA] * _NBUF_A,
            [pltpu.SemaphoreType.DMA] * _NBUF_A,
        ],
        compiler_params=cp,
    )(wt, weight[_TAIL_C * 128:].reshape(32, 128))

    out5 = pl.kernel(
        _lookup_body,
        out_type=jax.ShapeDtypeStruct((hist, dim, batch), jnp.float32),
        mesh=mesh,
        scratch_types=[
            pltpu.VMEM((25600,), jnp.int32),
            pltpu.VMEM((_NBUF_B, 128), jnp.int32),
            pltpu.VMEM((_NBUF_B, 128), jnp.int32),
            pltpu.VMEM((_NBUF_B, 128, 128), jnp.float32),
            pltpu.VMEM((_NBUF_B, dim, 128), jnp.float32),
            [pltpu.SemaphoreType.DMA] * _NBUF_B,
            [pltpu.SemaphoreType.DMA] * _NBUF_B,
        ],
        compiler_params=cp,
    )(idx, wrm)

    return out5.transpose(2, 0, 1)
